# baseline (speedup 1.0000x reference)
import functools

import jax
import jax.numpy as jnp
from jax import lax
from jax.experimental import pallas as pl
from jax.experimental.pallas import tpu as pltpu
from jax.experimental.pallas import tpu_sc as plsc

_INFO = plsc.get_sparse_core_info()
_NC = _INFO.num_cores
_NS = _INFO.num_subcores
_L = _INFO.num_lanes
_NW = _NC * _NS


def kernel(indices, items):
    B = indices.shape[0]
    V, D = items.shape
    b_per_w = B // _NW

    pairs = items.reshape(V // 2, 2 * D)

    mesh = plsc.VectorSubcoreMesh(core_axis_name="c", subcore_axis_name="s")

    @functools.partial(
        pl.kernel,
        mesh=mesh,
        out_type=jax.ShapeDtypeStruct((B // 2, 2 * D), jnp.float32),
        scratch_types=[
            pltpu.VMEM((b_per_w,), jnp.float32),
            pltpu.VMEM((b_per_w,), jnp.int32),
            pltpu.VMEM((b_per_w,), jnp.int32),
            pltpu.VMEM((b_per_w, 2 * D), jnp.float32),
            pltpu.VMEM((b_per_w // 2, 2 * D), jnp.float32),
            pltpu.SemaphoreType.DMA,
        ],
        compiler_params=pltpu.CompilerParams(needs_layout_passes=False),
    )
    def _gather(ind_hbm, pairs_hbm, out_hbm, find_v, pid_v, off_v, rows_v,
                sel_v, sem):
        wid = lax.axis_index("s") * _NC + lax.axis_index("c")
        base = wid * b_per_w
        pltpu.sync_copy(ind_hbm.at[pl.ds(base, b_per_w)], find_v)

        def conv(i, carry):
            x = find_v[pl.ds(i * _L, _L)]
            x = jnp.minimum(jnp.maximum(x, 0.0), 1.0) * jnp.float32(V)
            t = jnp.minimum(x.astype(jnp.int32), V - 1)
            pid_v[pl.ds(i * _L, _L)] = t >> 1
            off_v[pl.ds(i * _L, _L)] = (t & 1) * D
            return carry

        lax.fori_loop(0, b_per_w // _L, conv, 0)

        pltpu.async_copy(pairs_hbm.at[pid_v], rows_v, sem).wait()

        lanes = lax.iota(jnp.int32, _L)

        def select(i, carry):
            row = jnp.full((_L,), i, dtype=jnp.int32)
            half = plsc.load_gather(off_v, [row])
            r = i >> 1
            cbase = (i & 1) * D
            for j in range(D // _L):
                cols = half + (lanes + j * _L)
                sel_v[r, pl.ds(cbase + j * _L, _L)] = plsc.load_gather(
                    rows_v, [row, cols])
            return carry

        lax.fori_loop(0, b_per_w, select, 0)

        out_pairs = out_hbm
        pltpu.sync_copy(sel_v, out_pairs.at[pl.ds(wid * (b_per_w // 2),
                                                  b_per_w // 2)])

    out = _gather(indices, pairs)
    return out.reshape(B, D)


# speedup vs baseline: 1.3056x; 1.3056x over previous
import functools

import jax
import jax.numpy as jnp
from jax import lax
from jax.experimental import pallas as pl
from jax.experimental.pallas import tpu as pltpu
from jax.experimental.pallas import tpu_sc as plsc

_INFO = plsc.get_sparse_core_info()
_NC = _INFO.num_cores
_NS = _INFO.num_subcores
_L = _INFO.num_lanes
_NW = _NC * _NS
_NCHUNK = 4


def kernel(indices, items):
    B = indices.shape[0]
    V, D = items.shape
    DP = 128
    b_per_w = B // _NW
    chunk = b_per_w // _NCHUNK

    items_pad = jnp.pad(items, ((0, 0), (0, DP - D)))

    mesh = plsc.VectorSubcoreMesh(core_axis_name="c", subcore_axis_name="s")

    @functools.partial(
        pl.kernel,
        mesh=mesh,
        out_type=jax.ShapeDtypeStruct((B, DP), jnp.float32),
        scratch_types=[
            pltpu.VMEM((b_per_w,), jnp.float32),
            pltpu.VMEM((_NCHUNK, 1, chunk), jnp.int32),
            pltpu.VMEM((b_per_w, DP), jnp.float32),
            pltpu.SemaphoreType.DMA,
            pltpu.SemaphoreType.DMA,
        ],
    )
    def _gather(ind_hbm, table_hbm, out_hbm, find_v, idx_v, rows_v, gsem,
                wsem):
        wid = lax.axis_index("s") * _NC + lax.axis_index("c")
        base = wid * b_per_w
        pltpu.sync_copy(ind_hbm.at[pl.ds(base, b_per_w)], find_v)

        def conv(i, carry):
            c, j = i // (chunk // _L), i % (chunk // _L)
            x = find_v[pl.ds(i * _L, _L)]
            x = jnp.minimum(jnp.maximum(x, 0.0), 1.0) * jnp.float32(V)
            t = jnp.minimum(x.astype(jnp.int32), V - 1)
            idx_v[c, 0, pl.ds(j * _L, _L)] = t
            return carry

        lax.fori_loop(0, b_per_w // _L, conv, 0)

        gathers = []
        for c in range(_NCHUNK):
            gathers.append(pltpu.async_copy(
                table_hbm.at[idx_v.at[c, 0]],
                rows_v.at[pl.ds(c * chunk, chunk)], gsem))
        writes = []
        for c in range(_NCHUNK):
            gathers[c].wait()
            writes.append(pltpu.async_copy(
                rows_v.at[pl.ds(c * chunk, chunk)],
                out_hbm.at[pl.ds(base + c * chunk, chunk)], wsem))
        for w in writes:
            w.wait()

    out_pad = _gather(indices, items_pad)
    return out_pad[:, :D]


# speedup vs baseline: 1.3173x; 1.0090x over previous
import functools

import jax
import jax.numpy as jnp
from jax import lax
from jax.experimental import pallas as pl
from jax.experimental.pallas import tpu as pltpu
from jax.experimental.pallas import tpu_sc as plsc

_INFO = plsc.get_sparse_core_info()
_NC = _INFO.num_cores
_NS = _INFO.num_subcores
_L = _INFO.num_lanes
_NW = _NC * _NS


def kernel(indices, items):
    B = indices.shape[0]
    V, D = items.shape
    DP = 128
    b_per_w = B // _NW

    items_pad = jnp.pad(items, ((0, 0), (0, DP - D)))

    mesh = plsc.VectorSubcoreMesh(core_axis_name="c", subcore_axis_name="s")

    @functools.partial(
        pl.kernel,
        mesh=mesh,
        out_type=jax.ShapeDtypeStruct((B, DP), jnp.float32),
        scratch_types=[
            pltpu.VMEM((b_per_w,), jnp.float32),
            pltpu.VMEM((b_per_w,), jnp.int32),
            pltpu.VMEM((b_per_w, DP), jnp.float32),
            pltpu.SemaphoreType.DMA,
        ],
    )
    def _gather(ind_hbm, table_hbm, out_hbm, find_v, idx_v, rows_v, sem):
        wid = lax.axis_index("s") * _NC + lax.axis_index("c")
        base = wid * b_per_w
        pltpu.sync_copy(ind_hbm.at[pl.ds(base, b_per_w)], find_v)

        def body(i, carry):
            x = find_v[pl.ds(i * _L, _L)]
            x = jnp.minimum(jnp.maximum(x, 0.0), 1.0) * jnp.float32(V)
            t = jnp.minimum(x.astype(jnp.int32), V - 1)
            idx_v[pl.ds(i * _L, _L)] = t
            return carry

        lax.fori_loop(0, b_per_w // _L, body, 0)

        pltpu.async_copy(table_hbm.at[idx_v], rows_v, sem).wait()
        pltpu.sync_copy(rows_v, out_hbm.at[pl.ds(base, b_per_w)])

    out_pad = _gather(indices, items_pad)
    return out_pad[:, :D]
